# Initial kernel scaffold; baseline (speedup 1.0000x reference)
import jax
import jax.numpy as jnp
from jax.experimental import pallas as pl


def kernel(x, edge_index, edge_attr, params):
    raise NotImplementedError("write your pallas kernel here")


# speedup vs baseline: 18.5115x; 18.5115x over previous
import functools
from math import sqrt

import jax
import jax.numpy as jnp
from jax import lax
from jax.experimental import pallas as pl
from jax.experimental.pallas import tpu as pltpu
from jax.experimental.pallas import tpu_sc as plsc

N = 10000
E = 320000
D = 128
DE = 64
HEADS = 4
HD = D // HEADS

NP = 10240
NW = 32
CH = 128
NCH = E // CH
ITERS = (NCH + NW - 1) // NW

BE = 512
BN = 1000

def _leaky(v):
    return jnp.where(v >= 0, v, 0.1 * v)


@functools.lru_cache(maxsize=None)
def _build_sc_gather():
    mesh = plsc.VectorSubcoreMesh(core_axis_name="c", subcore_axis_name="s")

    @functools.partial(
        pl.kernel,
        out_type=(
            jax.ShapeDtypeStruct((E, D), jnp.float32),
            jax.ShapeDtypeStruct((E, D), jnp.float32),
        ),
        mesh=mesh,
        scratch_types=[
            pltpu.VMEM((CH,), jnp.int32),
            pltpu.VMEM((CH,), jnp.int32),
            pltpu.VMEM((CH, D), jnp.float32),
            pltpu.VMEM((CH, D), jnp.float32),
            pltpu.SemaphoreType.DMA,
            pltpu.SemaphoreType.DMA,
        ],
    )
    def sc_gather(x_hbm, src_hbm, dst_hbm, xs_hbm, xd_hbm,
                  idx_s, idx_d, buf_s, buf_d, sem_s, sem_d):
        wid = lax.axis_index("s") * 2 + lax.axis_index("c")

        def body(i, carry):
            cid = wid + i * NW

            @pl.when(cid < NCH)
            def _():
                base = cid * CH
                pltpu.sync_copy(src_hbm.at[pl.ds(base, CH)], idx_s)
                pltpu.sync_copy(dst_hbm.at[pl.ds(base, CH)], idx_d)
                a = pltpu.async_copy(x_hbm.at[idx_s], buf_s, sem_s)
                b = pltpu.async_copy(x_hbm.at[idx_d], buf_d, sem_d)
                a.wait()
                b.wait()
                pltpu.sync_copy(buf_s, xs_hbm.at[pl.ds(base, CH)])
                pltpu.sync_copy(buf_d, xd_hbm.at[pl.ds(base, CH)])

            return carry

        lax.fori_loop(0, ITERS, body, 0)

    return sc_gather


def _sc_gather(cx, src, dst):
    return _build_sc_gather()(cx, src, dst)


@functools.lru_cache(maxsize=None)
def _build_sc_scatter():
    mesh = plsc.VectorSubcoreMesh(core_axis_name="c", subcore_axis_name="s")

    @functools.partial(
        pl.kernel,
        out_type=jax.ShapeDtypeStruct((2 * NP, D), jnp.float32),
        mesh=mesh,
        scratch_types=[
            pltpu.VMEM((CH,), jnp.int32),
            pltpu.VMEM((CH, D), jnp.float32),
            pltpu.VMEM_SHARED((NP, D), jnp.float32),
        ],
    )
    def sc_scatter(comb_hbm, dst_hbm, z_hbm, acc_out, idx_v, buf, acc_sh):
        c = lax.axis_index("c")
        s = lax.axis_index("s")
        wid = s * 2 + c
        rows = NP // 16
        nk = rows // CH

        pltpu.sync_copy(z_hbm, buf)
        for k in range(nk):
            off = s * rows + k * CH
            pltpu.sync_copy(buf, acc_sh.at[pl.ds(off, CH)])
        plsc.subcore_barrier()

        def body(i, carry):
            cid = wid + i * NW

            @pl.when(cid < NCH)
            def _():
                base = cid * CH
                pltpu.sync_copy(dst_hbm.at[pl.ds(base, CH)], idx_v)
                pltpu.sync_copy(comb_hbm.at[pl.ds(base, CH)], buf)
                pltpu.sync_copy(buf, acc_sh.at[idx_v], add=True)

            return carry

        lax.fori_loop(0, ITERS, body, 0)
        plsc.subcore_barrier()

        for k in range(nk):
            off = s * rows + k * CH
            pltpu.sync_copy(acc_sh.at[pl.ds(off, CH)], buf)
            pltpu.sync_copy(buf, acc_out.at[pl.ds(c * NP + off, CH)])

    return sc_scatter


def _sc_scatter(rows128, dst, z):
    return _build_sc_scatter()(rows128, dst, z)


def _full(shape):
    return pl.BlockSpec(shape, lambda i: (0, 0))


def _rows(block):
    return pl.BlockSpec(block, lambda i: (i, 0))


def _node_encoder(x, w1, b1, w2, b2):
    def body(x_r, w1_r, b1_r, w2_r, b2_r, o_r):
        h = jnp.dot(x_r[...], w1_r[...], preferred_element_type=jnp.float32) + b1_r[...]
        o_r[...] = jnp.dot(h, w2_r[...], preferred_element_type=jnp.float32) + b2_r[...]

    return pl.pallas_call(
        body,
        grid=(N // BN,),
        in_specs=[_rows((BN, D)), _full((D, D)), _full((1, D)),
                  _full((D, D)), _full((1, D))],
        out_specs=_rows((BN, D)),
        out_shape=jax.ShapeDtypeStruct((N, D), jnp.float32),
    )(x, w1, b1.reshape(1, -1), w2, b2.reshape(1, -1))


def _edge_encoder(ea, w1, b1, w2, b2):
    def body(ea_r, w1_r, b1_r, w2_r, b2_r, o_r):
        h = _leaky(jnp.dot(ea_r[...], w1_r[...], preferred_element_type=jnp.float32) + b1_r[...])
        o_r[...] = jnp.dot(h, w2_r[...], preferred_element_type=jnp.float32) + b2_r[...]

    return pl.pallas_call(
        body,
        grid=(E // BE,),
        in_specs=[_rows((BE, 16)), _full((16, D)), _full((1, D)),
                  _full((D, DE)), _full((1, DE))],
        out_specs=_rows((BE, DE)),
        out_shape=jax.ShapeDtypeStruct((E, DE), jnp.float32),
    )(ea, w1, b1.reshape(1, -1), w2, b2.reshape(1, -1))


def _edge_stage_b(xs, xd, e, init_e, lw, sel):
    scale = 1.0 / sqrt(HD)

    def body(xs_r, xd_r, e_r, ie_r, wes_r, wed_r, wee_r, be_r, wq_r, wk_r,
             web_r, sel_r, ne_o, lg_o, gm_o, gm_s):
        i = pl.program_id(0)
        pre = (jnp.dot(xs_r[...], wes_r[...], preferred_element_type=jnp.float32)
               + jnp.dot(xd_r[...], wed_r[...], preferred_element_type=jnp.float32)
               + jnp.dot(e_r[...], wee_r[...], preferred_element_type=jnp.float32)
               + be_r[...])
        ne = _leaky(pre) + ie_r[...]
        ne_o[...] = ne
        qd = jnp.dot(xd_r[...], wq_r[...], preferred_element_type=jnp.float32)
        ks = jnp.dot(xs_r[...], wk_r[...], preferred_element_type=jnp.float32)
        lg = (jnp.dot(qd * ks, sel_r[...], preferred_element_type=jnp.float32) * scale
              + jnp.dot(ne, web_r[...], preferred_element_type=jnp.float32))
        lg_o[...] = lg
        bm = jnp.max(lg)
        prev = jnp.where(i == 0, jnp.float32(-1e30), gm_s[0])
        gm_s[0] = jnp.maximum(prev, bm)
        gm_o[0, 0] = gm_s[0]

    return pl.pallas_call(
        body,
        grid=(E // BE,),
        in_specs=[_rows((BE, D)), _rows((BE, D)), _rows((BE, DE)),
                  _rows((BE, DE)), _full((D, DE)), _full((D, DE)),
                  _full((DE, DE)), _full((1, DE)), _full((D, D)),
                  _full((D, D)), _full((DE, HEADS)), _full((D, HEADS))],
        out_specs=(_rows((BE, DE)), _rows((BE, HEADS)),
                   pl.BlockSpec((1, 1), lambda i: (0, 0),
                                memory_space=pltpu.SMEM)),
        out_shape=(jax.ShapeDtypeStruct((E, DE), jnp.float32),
                   jax.ShapeDtypeStruct((E, HEADS), jnp.float32),
                   jax.ShapeDtypeStruct((1, 1), jnp.float32)),
        scratch_shapes=[pltpu.SMEM((1,), jnp.float32)],
    )(xs, xd, e, init_e, lw['we'][:D], lw['we'][D:2 * D], lw['we'][2 * D:],
      lw['be'].reshape(1, -1), lw['wq'], lw['wk'], lw['web'], sel)


def _edge_stage_c(xs, new_e, logits, gmax, lw, rep4, p4to128):
    def body(xs_r, ne_r, lg_r, gm_r, wmx_r, wme_r, bm_r, rep_r, p128_r,
             em_o, ep_o):
        pre = (jnp.dot(xs_r[...], wmx_r[...], preferred_element_type=jnp.float32)
               + jnp.dot(ne_r[...], wme_r[...], preferred_element_type=jnp.float32)
               + bm_r[...])
        msg = _leaky(pre)
        ex = jnp.exp(lg_r[...] - gm_r[0, 0] + 40.0)
        exb = jnp.dot(ex, rep_r[...], preferred_element_type=jnp.float32)
        em_o[...] = exb * msg
        ep_o[...] = jnp.dot(ex, p128_r[...], preferred_element_type=jnp.float32)

    return pl.pallas_call(
        body,
        grid=(E // BE,),
        in_specs=[_rows((BE, D)), _rows((BE, DE)), _rows((BE, HEADS)),
                  pl.BlockSpec((1, 1), lambda i: (0, 0),
                               memory_space=pltpu.SMEM),
                  _full((D, D)), _full((DE, D)), _full((1, D)),
                  _full((HEADS, D)), _full((HEADS, D))],
        out_specs=(_rows((BE, D)), _rows((BE, D))),
        out_shape=(jax.ShapeDtypeStruct((E, D), jnp.float32),
                   jax.ShapeDtypeStruct((E, D), jnp.float32)),
    )(xs, new_e, logits, gmax, lw['wm'][:D], lw['wm'][D:],
      lw['bm'].reshape(1, -1), rep4, p4to128)


def _node_stage(x, init_x, num0, num1, den0, den1, lw, rep128, dec=None):
    has_dec = dec is not None

    def body(*refs):
        if has_dec:
            (x_r, ix_r, n0_r, n1_r, d0_r, d1_r, wnx_r, wna_r, bn_r, rep_r,
             dw1_r, db1_r, dw2_r, db2_r, dw3_r, db3_r, o_r) = refs
        else:
            (x_r, ix_r, n0_r, n1_r, d0_r, d1_r, wnx_r, wna_r, bn_r, rep_r,
             o_r) = refs
        den = jnp.dot(d0_r[...] + d1_r[...], rep_r[...],
                      preferred_element_type=jnp.float32)
        num = n0_r[...] + n1_r[...]
        agg = jnp.where(den > 0, num / den, 0.0)
        pre = (jnp.dot(x_r[...], wnx_r[...], preferred_element_type=jnp.float32)
               + jnp.dot(agg, wna_r[...], preferred_element_type=jnp.float32)
               + bn_r[...])
        nx = _leaky(pre) + ix_r[...]
        if has_dec:
            h = _leaky(jnp.dot(nx, dw1_r[...], preferred_element_type=jnp.float32) + db1_r[...])
            h = _leaky(jnp.dot(h, dw2_r[...], preferred_element_type=jnp.float32) + db2_r[...])
            o_r[...] = jnp.dot(h, dw3_r[...], preferred_element_type=jnp.float32) + db3_r[...]
        else:
            o_r[...] = nx

    ins = [_rows((BN, D)), _rows((BN, D)), _rows((BN, D)), _rows((BN, D)),
           _rows((BN, D)), _rows((BN, D)), _full((D, D)), _full((D, D)),
           _full((1, D)), _full((D, D))]
    args = [x, init_x, num0, num1, den0, den1, lw['wn'][:D], lw['wn'][D:],
            lw['bn'].reshape(1, -1), rep128]
    if has_dec:
        ins += [_full((D, D)), _full((1, D)), _full((D, D)), _full((1, D)),
                _full((D, 1)), _full((1, 1))]
        args += [dec['w1'], dec['b1'].reshape(1, -1), dec['w2'],
                 dec['b2'].reshape(1, -1), dec['w3'], dec['b3'].reshape(1, -1)]
        out_spec = _rows((BN, 1))
        out_shape = jax.ShapeDtypeStruct((N, 1), jnp.float32)
    else:
        out_spec = _rows((BN, D))
        out_shape = jax.ShapeDtypeStruct((N, D), jnp.float32)

    return pl.pallas_call(
        body,
        grid=(N // BN,),
        in_specs=ins,
        out_specs=out_spec,
        out_shape=out_shape,
    )(*args)


def kernel(x, edge_index, edge_attr, params):
    p = params
    src = edge_index[0].astype(jnp.int32)
    dst = edge_index[1].astype(jnp.int32)

    hd_ids = jnp.arange(D, dtype=jnp.int32) // HD
    sel = (hd_ids[:, None] == jnp.arange(HEADS)[None, :]).astype(jnp.float32)
    rep4 = sel.T
    p4to128 = jnp.eye(HEADS, D, dtype=jnp.float32)
    rep128 = jnp.zeros((D, D), jnp.float32).at[:HEADS].set(rep4)
    zc = jnp.zeros((CH, D), jnp.float32)

    init_x = _node_encoder(x, p['ne_w1'], p['ne_b1'], p['ne_w2'], p['ne_b2'])
    init_e = _edge_encoder(edge_attr, p['ee_w1'], p['ee_b1'],
                           p['ee_w2'], p['ee_b2'])

    cx, ce = init_x, init_e
    dec = {'w1': p['dec_w1'], 'b1': p['dec_b1'], 'w2': p['dec_w2'],
           'b2': p['dec_b2'], 'w3': p['dec_w3'], 'b3': p['dec_b3']}

    for li, lw in enumerate(p['layers']):
        xs, xd = _sc_gather(cx, src, dst)
        new_e, logits, gmax = _edge_stage_b(xs, xd, ce, init_e, lw, sel)
        exmsg, expad = _edge_stage_c(xs, new_e, logits, gmax, lw, rep4,
                                     p4to128)
        num = _sc_scatter(exmsg, dst, zc)
        den = _sc_scatter(expad, dst, zc)
        last = li == len(p['layers']) - 1
        out = _node_stage(cx, init_x, num[:N], num[NP:NP + N], den[:N],
                          den[NP:NP + N], lw, rep128,
                          dec=dec if last else None)
        if last:
            return out
        cx, ce = out, new_e


# speedup vs baseline: 20.6408x; 1.1150x over previous
import functools
from math import sqrt

import jax
import jax.numpy as jnp
from jax import lax
from jax.experimental import pallas as pl
from jax.experimental.pallas import tpu as pltpu
from jax.experimental.pallas import tpu_sc as plsc

N = 10000
E = 320000
D = 128
DE = 64
HEADS = 4
HD = D // HEADS

NP = 10240
NW = 32
CH = 128
NCH = E // CH
ITERS = (NCH + NW - 1) // NW

BE = 512
BN = 1000

def _leaky(v):
    return jnp.where(v >= 0, v, 0.1 * v)


@functools.lru_cache(maxsize=None)
def _build_sc_gather():
    mesh = plsc.VectorSubcoreMesh(core_axis_name="c", subcore_axis_name="s")

    @functools.partial(
        pl.kernel,
        out_type=(
            jax.ShapeDtypeStruct((E, D), jnp.float32),
            jax.ShapeDtypeStruct((E, D), jnp.float32),
        ),
        mesh=mesh,
        scratch_types=[
            pltpu.VMEM((4, CH), jnp.int32),
            pltpu.VMEM((4 * CH, D), jnp.float32),
            pltpu.SemaphoreType.DMA,
            pltpu.SemaphoreType.DMA,
            pltpu.SemaphoreType.DMA,
            pltpu.SemaphoreType.DMA,
        ],
    )
    def sc_gather(x_hbm, src_hbm, dst_hbm, xs_hbm, xd_hbm,
                  idx_v, bufs, gsem0, gsem1, wsem0, wsem1):
        wid = lax.axis_index("s") * 2 + lax.axis_index("c")
        gsem = (gsem0, gsem1)
        wsem = (wsem0, wsem1)

        def slot_refs(b):
            return (idx_v.at[b], idx_v.at[2 + b],
                    bufs.at[pl.ds(b * CH, CH)],
                    bufs.at[pl.ds((2 + b) * CH, CH)])

        def body(j, carry):
            for b in range(2):
                i = 2 * j + b
                isl, idl, bsl, bdl = slot_refs(b)
                iso, ido, bso, bdo = slot_refs(1 - b)
                cid_i = wid + i * NW
                cid_p = cid_i - NW
                cid_pp = cid_i - 2 * NW

                @pl.when((i >= 2) & (cid_pp >= 0) & (cid_pp < NCH))
                def _():
                    base = cid_pp * CH
                    pltpu.make_async_copy(
                        bsl, xs_hbm.at[pl.ds(base, CH)], wsem[b]).wait()
                    pltpu.make_async_copy(
                        bdl, xd_hbm.at[pl.ds(base, CH)], wsem[b]).wait()

                @pl.when(cid_i < NCH)
                def _():
                    base = cid_i * CH
                    pltpu.sync_copy(src_hbm.at[pl.ds(base, CH)], isl)
                    pltpu.sync_copy(dst_hbm.at[pl.ds(base, CH)], idl)
                    pltpu.async_copy(x_hbm.at[isl], bsl, gsem[b])
                    pltpu.async_copy(x_hbm.at[idl], bdl, gsem[b])

                @pl.when((i >= 1) & (cid_p >= 0) & (cid_p < NCH))
                def _():
                    base = cid_p * CH
                    pltpu.make_async_copy(
                        x_hbm.at[iso], bso, gsem[1 - b]).wait()
                    pltpu.make_async_copy(
                        x_hbm.at[ido], bdo, gsem[1 - b]).wait()
                    pltpu.async_copy(
                        bso, xs_hbm.at[pl.ds(base, CH)], wsem[1 - b])
                    pltpu.async_copy(
                        bdo, xd_hbm.at[pl.ds(base, CH)], wsem[1 - b])

            return carry

        lax.fori_loop(0, (ITERS + 2 + 1) // 2, body, 0)

    return sc_gather


def _sc_gather(cx, src, dst):
    return _build_sc_gather()(cx, src, dst)


@functools.lru_cache(maxsize=None)
def _build_sc_scatter():
    mesh = plsc.VectorSubcoreMesh(core_axis_name="c", subcore_axis_name="s")

    @functools.partial(
        pl.kernel,
        out_type=jax.ShapeDtypeStruct((2 * NP, D), jnp.float32),
        mesh=mesh,
        scratch_types=[
            pltpu.VMEM((2, CH), jnp.int32),
            pltpu.VMEM((2 * CH, D), jnp.float32),
            pltpu.VMEM_SHARED((NP, D), jnp.float32),
            pltpu.SemaphoreType.DMA,
            pltpu.SemaphoreType.DMA,
            pltpu.SemaphoreType.DMA,
            pltpu.SemaphoreType.DMA,
        ],
    )
    def sc_scatter(comb_hbm, dst_hbm, z_hbm, acc_out, idx_v, bufs, acc_sh,
                   dsem0, dsem1, asem0, asem1):
        c = lax.axis_index("c")
        s = lax.axis_index("s")
        wid = s * 2 + c
        rows = NP // 16
        nk = rows // CH
        dsem = (dsem0, dsem1)
        asem = (asem0, asem1)

        pltpu.sync_copy(z_hbm, bufs.at[pl.ds(0, CH)])
        for k in range(nk):
            off = s * rows + k * CH
            pltpu.sync_copy(bufs.at[pl.ds(0, CH)], acc_sh.at[pl.ds(off, CH)])
        plsc.subcore_barrier()

        def body(j, carry):
            for b in range(2):
                i = 2 * j + b
                isl = idx_v.at[b]
                bsl = bufs.at[pl.ds(b * CH, CH)]
                iso = idx_v.at[1 - b]
                bso = bufs.at[pl.ds((1 - b) * CH, CH)]
                cid_i = wid + i * NW
                cid_p = cid_i - NW
                cid_pp = cid_i - 2 * NW

                @pl.when((cid_pp >= 0) & (cid_pp < NCH))
                def _():
                    pltpu.make_async_copy(
                        bsl, acc_sh.at[isl], asem[b]).wait()

                @pl.when(cid_i < NCH)
                def _():
                    base = cid_i * CH
                    pltpu.sync_copy(dst_hbm.at[pl.ds(base, CH)], isl)
                    pltpu.async_copy(
                        comb_hbm.at[pl.ds(base, CH)], bsl, dsem[b])

                @pl.when((cid_p >= 0) & (cid_p < NCH))
                def _():
                    base = cid_p * CH
                    pltpu.make_async_copy(
                        comb_hbm.at[pl.ds(base, CH)], bso, dsem[1 - b]).wait()
                    pltpu.async_copy(
                        bso, acc_sh.at[iso], asem[1 - b], add=True)

            return carry

        lax.fori_loop(0, (ITERS + 2 + 1) // 2, body, 0)
        plsc.subcore_barrier()

        for k in range(nk):
            off = s * rows + k * CH
            pltpu.sync_copy(acc_sh.at[pl.ds(off, CH)],
                            bufs.at[pl.ds(0, CH)])
            pltpu.sync_copy(bufs.at[pl.ds(0, CH)],
                            acc_out.at[pl.ds(c * NP + off, CH)])

    return sc_scatter


def _sc_scatter(rows128, dst, z):
    return _build_sc_scatter()(rows128, dst, z)


def _full(shape):
    return pl.BlockSpec(shape, lambda i: (0, 0))


def _rows(block):
    return pl.BlockSpec(block, lambda i: (i, 0))


def _node_encoder(x, w1, b1, w2, b2):
    def body(x_r, w1_r, b1_r, w2_r, b2_r, o_r):
        h = jnp.dot(x_r[...], w1_r[...], preferred_element_type=jnp.float32) + b1_r[...]
        o_r[...] = jnp.dot(h, w2_r[...], preferred_element_type=jnp.float32) + b2_r[...]

    return pl.pallas_call(
        body,
        grid=(N // BN,),
        in_specs=[_rows((BN, D)), _full((D, D)), _full((1, D)),
                  _full((D, D)), _full((1, D))],
        out_specs=_rows((BN, D)),
        out_shape=jax.ShapeDtypeStruct((N, D), jnp.float32),
    )(x, w1, b1.reshape(1, -1), w2, b2.reshape(1, -1))


def _edge_encoder(ea, w1, b1, w2, b2):
    def body(ea_r, w1_r, b1_r, w2_r, b2_r, o_r):
        h = _leaky(jnp.dot(ea_r[...], w1_r[...], preferred_element_type=jnp.float32) + b1_r[...])
        o_r[...] = jnp.dot(h, w2_r[...], preferred_element_type=jnp.float32) + b2_r[...]

    return pl.pallas_call(
        body,
        grid=(E // BE,),
        in_specs=[_rows((BE, 16)), _full((16, D)), _full((1, D)),
                  _full((D, DE)), _full((1, DE))],
        out_specs=_rows((BE, DE)),
        out_shape=jax.ShapeDtypeStruct((E, DE), jnp.float32),
    )(ea, w1, b1.reshape(1, -1), w2, b2.reshape(1, -1))


def _edge_stage_b(xs, xd, e, init_e, lw, sel):
    scale = 1.0 / sqrt(HD)

    def body(xs_r, xd_r, e_r, ie_r, wes_r, wed_r, wee_r, be_r, wq_r, wk_r,
             web_r, sel_r, ne_o, lg_o, gm_o, gm_s):
        i = pl.program_id(0)
        pre = (jnp.dot(xs_r[...], wes_r[...], preferred_element_type=jnp.float32)
               + jnp.dot(xd_r[...], wed_r[...], preferred_element_type=jnp.float32)
               + jnp.dot(e_r[...], wee_r[...], preferred_element_type=jnp.float32)
               + be_r[...])
        ne = _leaky(pre) + ie_r[...]
        ne_o[...] = ne
        qd = jnp.dot(xd_r[...], wq_r[...], preferred_element_type=jnp.float32)
        ks = jnp.dot(xs_r[...], wk_r[...], preferred_element_type=jnp.float32)
        lg = (jnp.dot(qd * ks, sel_r[...], preferred_element_type=jnp.float32) * scale
              + jnp.dot(ne, web_r[...], preferred_element_type=jnp.float32))
        lg_o[...] = lg
        bm = jnp.max(lg)
        prev = jnp.where(i == 0, jnp.float32(-1e30), gm_s[0])
        gm_s[0] = jnp.maximum(prev, bm)
        gm_o[0, 0] = gm_s[0]

    return pl.pallas_call(
        body,
        grid=(E // BE,),
        in_specs=[_rows((BE, D)), _rows((BE, D)), _rows((BE, DE)),
                  _rows((BE, DE)), _full((D, DE)), _full((D, DE)),
                  _full((DE, DE)), _full((1, DE)), _full((D, D)),
                  _full((D, D)), _full((DE, HEADS)), _full((D, HEADS))],
        out_specs=(_rows((BE, DE)), _rows((BE, HEADS)),
                   pl.BlockSpec((1, 1), lambda i: (0, 0),
                                memory_space=pltpu.SMEM)),
        out_shape=(jax.ShapeDtypeStruct((E, DE), jnp.float32),
                   jax.ShapeDtypeStruct((E, HEADS), jnp.float32),
                   jax.ShapeDtypeStruct((1, 1), jnp.float32)),
        scratch_shapes=[pltpu.SMEM((1,), jnp.float32)],
    )(xs, xd, e, init_e, lw['we'][:D], lw['we'][D:2 * D], lw['we'][2 * D:],
      lw['be'].reshape(1, -1), lw['wq'], lw['wk'], lw['web'], sel)


def _edge_stage_c(xs, new_e, logits, gmax, lw, rep4, p4to128):
    def body(xs_r, ne_r, lg_r, gm_r, wmx_r, wme_r, bm_r, rep_r, p128_r,
             em_o, ep_o):
        pre = (jnp.dot(xs_r[...], wmx_r[...], preferred_element_type=jnp.float32)
               + jnp.dot(ne_r[...], wme_r[...], preferred_element_type=jnp.float32)
               + bm_r[...])
        msg = _leaky(pre)
        ex = jnp.exp(lg_r[...] - gm_r[0, 0] + 40.0)
        exb = jnp.dot(ex, rep_r[...], preferred_element_type=jnp.float32)
        em_o[...] = exb * msg
        ep_o[...] = jnp.dot(ex, p128_r[...], preferred_element_type=jnp.float32)

    return pl.pallas_call(
        body,
        grid=(E // BE,),
        in_specs=[_rows((BE, D)), _rows((BE, DE)), _rows((BE, HEADS)),
                  pl.BlockSpec((1, 1), lambda i: (0, 0),
                               memory_space=pltpu.SMEM),
                  _full((D, D)), _full((DE, D)), _full((1, D)),
                  _full((HEADS, D)), _full((HEADS, D))],
        out_specs=(_rows((BE, D)), _rows((BE, D))),
        out_shape=(jax.ShapeDtypeStruct((E, D), jnp.float32),
                   jax.ShapeDtypeStruct((E, D), jnp.float32)),
    )(xs, new_e, logits, gmax, lw['wm'][:D], lw['wm'][D:],
      lw['bm'].reshape(1, -1), rep4, p4to128)


def _node_stage(x, init_x, num0, num1, den0, den1, lw, rep128, dec=None):
    has_dec = dec is not None

    def body(*refs):
        if has_dec:
            (x_r, ix_r, n0_r, n1_r, d0_r, d1_r, wnx_r, wna_r, bn_r, rep_r,
             dw1_r, db1_r, dw2_r, db2_r, dw3_r, db3_r, o_r) = refs
        else:
            (x_r, ix_r, n0_r, n1_r, d0_r, d1_r, wnx_r, wna_r, bn_r, rep_r,
             o_r) = refs
        den = jnp.dot(d0_r[...] + d1_r[...], rep_r[...],
                      preferred_element_type=jnp.float32)
        num = n0_r[...] + n1_r[...]
        agg = jnp.where(den > 0, num / den, 0.0)
        pre = (jnp.dot(x_r[...], wnx_r[...], preferred_element_type=jnp.float32)
               + jnp.dot(agg, wna_r[...], preferred_element_type=jnp.float32)
               + bn_r[...])
        nx = _leaky(pre) + ix_r[...]
        if has_dec:
            h = _leaky(jnp.dot(nx, dw1_r[...], preferred_element_type=jnp.float32) + db1_r[...])
            h = _leaky(jnp.dot(h, dw2_r[...], preferred_element_type=jnp.float32) + db2_r[...])
            o_r[...] = jnp.dot(h, dw3_r[...], preferred_element_type=jnp.float32) + db3_r[...]
        else:
            o_r[...] = nx

    ins = [_rows((BN, D)), _rows((BN, D)), _rows((BN, D)), _rows((BN, D)),
           _rows((BN, D)), _rows((BN, D)), _full((D, D)), _full((D, D)),
           _full((1, D)), _full((D, D))]
    args = [x, init_x, num0, num1, den0, den1, lw['wn'][:D], lw['wn'][D:],
            lw['bn'].reshape(1, -1), rep128]
    if has_dec:
        ins += [_full((D, D)), _full((1, D)), _full((D, D)), _full((1, D)),
                _full((D, 1)), _full((1, 1))]
        args += [dec['w1'], dec['b1'].reshape(1, -1), dec['w2'],
                 dec['b2'].reshape(1, -1), dec['w3'], dec['b3'].reshape(1, -1)]
        out_spec = _rows((BN, 1))
        out_shape = jax.ShapeDtypeStruct((N, 1), jnp.float32)
    else:
        out_spec = _rows((BN, D))
        out_shape = jax.ShapeDtypeStruct((N, D), jnp.float32)

    return pl.pallas_call(
        body,
        grid=(N // BN,),
        in_specs=ins,
        out_specs=out_spec,
        out_shape=out_shape,
    )(*args)


def kernel(x, edge_index, edge_attr, params):
    p = params
    src = edge_index[0].astype(jnp.int32)
    dst = edge_index[1].astype(jnp.int32)

    hd_ids = jnp.arange(D, dtype=jnp.int32) // HD
    sel = (hd_ids[:, None] == jnp.arange(HEADS)[None, :]).astype(jnp.float32)
    rep4 = sel.T
    p4to128 = jnp.eye(HEADS, D, dtype=jnp.float32)
    rep128 = jnp.zeros((D, D), jnp.float32).at[:HEADS].set(rep4)
    zc = jnp.zeros((CH, D), jnp.float32)

    init_x = _node_encoder(x, p['ne_w1'], p['ne_b1'], p['ne_w2'], p['ne_b2'])
    init_e = _edge_encoder(edge_attr, p['ee_w1'], p['ee_b1'],
                           p['ee_w2'], p['ee_b2'])

    cx, ce = init_x, init_e
    dec = {'w1': p['dec_w1'], 'b1': p['dec_b1'], 'w2': p['dec_w2'],
           'b2': p['dec_b2'], 'w3': p['dec_w3'], 'b3': p['dec_b3']}

    for li, lw in enumerate(p['layers']):
        xs, xd = _sc_gather(cx, src, dst)
        new_e, logits, gmax = _edge_stage_b(xs, xd, ce, init_e, lw, sel)
        exmsg, expad = _edge_stage_c(xs, new_e, logits, gmax, lw, rep4,
                                     p4to128)
        num = _sc_scatter(exmsg, dst, zc)
        den = _sc_scatter(expad, dst, zc)
        last = li == len(p['layers']) - 1
        out = _node_stage(cx, init_x, num[:N], num[NP:NP + N], den[:N],
                          den[NP:NP + N], lw, rep128,
                          dec=dec if last else None)
        if last:
            return out
        cx, ce = out, new_e


# speedup vs baseline: 20.9006x; 1.0126x over previous
import functools
from math import sqrt

import jax
import jax.numpy as jnp
from jax import lax
from jax.experimental import pallas as pl
from jax.experimental.pallas import tpu as pltpu
from jax.experimental.pallas import tpu_sc as plsc

N = 10000
E = 320000
D = 128
DE = 64
HEADS = 4
HD = D // HEADS

NP = 10240
NW = 32
CH = 128
G = 256
JG = E // G
NJS = E // CH

BE = 512
BN = 1000

def _leaky(v):
    return jnp.where(v >= 0, v, 0.1 * v)


@functools.lru_cache(maxsize=None)
def _build_sc_gather():
    mesh = plsc.VectorSubcoreMesh(core_axis_name="c", subcore_axis_name="s")

    @functools.partial(
        pl.kernel,
        out_type=jax.ShapeDtypeStruct((2 * E, D), jnp.float32),
        mesh=mesh,
        scratch_types=[
            pltpu.VMEM((2, 2, CH), jnp.int32),
            pltpu.VMEM((2 * G, D), jnp.float32),
            pltpu.SemaphoreType.DMA,
            pltpu.SemaphoreType.DMA,
            pltpu.SemaphoreType.DMA,
            pltpu.SemaphoreType.DMA,
        ],
    )
    def sc_gather(x_hbm, sd_hbm, out_hbm, idx_v, bufs,
                  gsem0, gsem1, wsem0, wsem1):
        c = lax.axis_index("c")
        s = lax.axis_index("s")
        jbase = c * JG
        gsem = (gsem0, gsem1)
        wsem = (wsem0, wsem1)

        def body(j, carry):
            for b in range(2):
                i = 2 * j + b
                l_i = s + i * 16
                l_p = l_i - 16
                l_pp = l_i - 32
                bsl = bufs.at[pl.ds(b * G, G)]
                bso = bufs.at[pl.ds((1 - b) * G, G)]

                @pl.when((l_pp >= 0) & (l_pp < JG))
                def _():
                    pltpu.make_async_copy(
                        bsl, out_hbm.at[pl.ds((jbase + l_pp) * G, G)],
                        wsem[b]).wait()

                @pl.when(l_i < JG)
                def _():
                    jid = jbase + l_i
                    pltpu.sync_copy(sd_hbm.at[pl.ds(jid * 2, 2)],
                                    idx_v.at[b])
                    pltpu.async_copy(x_hbm.at[idx_v.at[b, 0]],
                                     bufs.at[pl.ds(b * G, CH)], gsem[b])
                    pltpu.async_copy(x_hbm.at[idx_v.at[b, 1]],
                                     bufs.at[pl.ds(b * G + CH, CH)], gsem[b])

                @pl.when((l_p >= 0) & (l_p < JG))
                def _():
                    jid = jbase + l_p
                    pltpu.make_async_copy(
                        x_hbm.at[idx_v.at[1 - b, 0]],
                        bufs.at[pl.ds((1 - b) * G, CH)], gsem[1 - b]).wait()
                    pltpu.make_async_copy(
                        x_hbm.at[idx_v.at[1 - b, 1]],
                        bufs.at[pl.ds((1 - b) * G + CH, CH)],
                        gsem[1 - b]).wait()
                    pltpu.async_copy(
                        bso, out_hbm.at[pl.ds(jid * G, G)], wsem[1 - b])

            return carry

        viters = (JG + 15) // 16 + 2
        lax.fori_loop(0, (viters + 1) // 2, body, 0)

    return sc_gather


def _sc_gather(cx, sd2d):
    return _build_sc_gather()(cx, sd2d)


@functools.lru_cache(maxsize=None)
def _build_sc_scatter():
    mesh = plsc.VectorSubcoreMesh(core_axis_name="c", subcore_axis_name="s")

    @functools.partial(
        pl.kernel,
        out_type=jax.ShapeDtypeStruct((2 * NP, D), jnp.float32),
        mesh=mesh,
        scratch_types=[
            pltpu.VMEM((2, CH), jnp.int32),
            pltpu.VMEM((2 * CH, D), jnp.float32),
            pltpu.VMEM_SHARED((NP, D), jnp.float32),
            pltpu.SemaphoreType.DMA,
            pltpu.SemaphoreType.DMA,
            pltpu.SemaphoreType.DMA,
            pltpu.SemaphoreType.DMA,
        ],
    )
    def sc_scatter(comb_hbm, sd_hbm, z_hbm, acc_out, idx_v, bufs, acc_sh,
                   dsem0, dsem1, asem0, asem1):
        c = lax.axis_index("c")
        s = lax.axis_index("s")
        wid = s * 2 + c
        rows = NP // 16
        nk = rows // CH
        dsem = (dsem0, dsem1)
        asem = (asem0, asem1)

        pltpu.sync_copy(z_hbm, bufs.at[pl.ds(0, CH)])
        for k in range(nk):
            off = s * rows + k * CH
            pltpu.sync_copy(bufs.at[pl.ds(0, CH)], acc_sh.at[pl.ds(off, CH)])
        plsc.subcore_barrier()

        def body(j, carry):
            for b in range(2):
                i = 2 * j + b
                l_i = wid + i * NW
                l_p = l_i - NW
                l_pp = l_i - 2 * NW
                bsl = bufs.at[pl.ds(b * CH, CH)]
                bso = bufs.at[pl.ds((1 - b) * CH, CH)]

                @pl.when((l_pp >= 0) & (l_pp < NJS))
                def _():
                    pltpu.make_async_copy(
                        bsl, acc_sh.at[idx_v.at[b]], asem[b]).wait()

                @pl.when(l_i < NJS)
                def _():
                    pltpu.sync_copy(sd_hbm.at[NJS + l_i], idx_v.at[b])
                    pltpu.async_copy(
                        comb_hbm.at[pl.ds(l_i * CH, CH)], bsl, dsem[b])

                @pl.when((l_p >= 0) & (l_p < NJS))
                def _():
                    pltpu.make_async_copy(
                        comb_hbm.at[pl.ds(l_p * CH, CH)], bso,
                        dsem[1 - b]).wait()
                    pltpu.async_copy(
                        bso, acc_sh.at[idx_v.at[1 - b]], asem[1 - b],
                        add=True)

            return carry

        viters = (NJS + NW - 1) // NW + 2
        lax.fori_loop(0, (viters + 1) // 2, body, 0)
        plsc.subcore_barrier()

        for k in range(nk):
            off = s * rows + k * CH
            pltpu.sync_copy(acc_sh.at[pl.ds(off, CH)],
                            bufs.at[pl.ds(0, CH)])
            pltpu.sync_copy(bufs.at[pl.ds(0, CH)],
                            acc_out.at[pl.ds(c * NP + off, CH)])

    return sc_scatter


def _sc_scatter(rows128, sd2d, z):
    return _build_sc_scatter()(rows128, sd2d, z)


def _full(shape):
    return pl.BlockSpec(shape, lambda i: (0, 0))


def _rows(block):
    return pl.BlockSpec(block, lambda i: (i, 0))


def _node_encoder(x, w1, b1, w2, b2):
    def body(x_r, w1_r, b1_r, w2_r, b2_r, o_r):
        h = jnp.dot(x_r[...], w1_r[...], preferred_element_type=jnp.float32) + b1_r[...]
        o_r[...] = jnp.dot(h, w2_r[...], preferred_element_type=jnp.float32) + b2_r[...]

    return pl.pallas_call(
        body,
        grid=(N // BN,),
        in_specs=[_rows((BN, D)), _full((D, D)), _full((1, D)),
                  _full((D, D)), _full((1, D))],
        out_specs=_rows((BN, D)),
        out_shape=jax.ShapeDtypeStruct((N, D), jnp.float32),
    )(x, w1, b1.reshape(1, -1), w2, b2.reshape(1, -1))


def _edge_encoder(ea, w1, b1, w2, b2):
    def body(ea_r, w1_r, b1_r, w2_r, b2_r, o_r):
        h = _leaky(jnp.dot(ea_r[...], w1_r[...], preferred_element_type=jnp.float32) + b1_r[...])
        o_r[...] = jnp.dot(h, w2_r[...], preferred_element_type=jnp.float32) + b2_r[...]

    return pl.pallas_call(
        body,
        grid=(E // BE,),
        in_specs=[_rows((BE, 16)), _full((16, D)), _full((1, D)),
                  _full((D, DE)), _full((1, DE))],
        out_specs=_rows((BE, DE)),
        out_shape=jax.ShapeDtypeStruct((E, DE), jnp.float32),
    )(ea, w1, b1.reshape(1, -1), w2, b2.reshape(1, -1))


_EB = E // BE


def _edge_stage_b(xsd, e, init_e, lw, sel):
    scale = 1.0 / sqrt(HD)

    def body(xs_r, xd_r, e_r, ie_r, wes_r, wed_r, wee_r, be_r, wq_r, wk_r,
             web_r, sel_r, ne_o, lg_o, gm_o, gm_s):
        i = pl.program_id(0)
        pre = (jnp.dot(xs_r[...], wes_r[...], preferred_element_type=jnp.float32)
               + jnp.dot(xd_r[...], wed_r[...], preferred_element_type=jnp.float32)
               + jnp.dot(e_r[...], wee_r[...], preferred_element_type=jnp.float32)
               + be_r[...])
        ne = _leaky(pre) + ie_r[...]
        ne_o[...] = ne
        qd = jnp.dot(xd_r[...], wq_r[...], preferred_element_type=jnp.float32)
        ks = jnp.dot(xs_r[...], wk_r[...], preferred_element_type=jnp.float32)
        lg = (jnp.dot(qd * ks, sel_r[...], preferred_element_type=jnp.float32) * scale
              + jnp.dot(ne, web_r[...], preferred_element_type=jnp.float32))
        lg_o[...] = lg
        bm = jnp.max(lg)
        prev = jnp.where(i == 0, jnp.float32(-1e30), gm_s[0])
        gm_s[0] = jnp.maximum(prev, bm)
        gm_o[0, 0] = gm_s[0]

    return pl.pallas_call(
        body,
        grid=(E // BE,),
        in_specs=[_rows((BE, D)),
                  pl.BlockSpec((BE, D), lambda i: (i + _EB, 0)),
                  _rows((BE, DE)),
                  _rows((BE, DE)), _full((D, DE)), _full((D, DE)),
                  _full((DE, DE)), _full((1, DE)), _full((D, D)),
                  _full((D, D)), _full((DE, HEADS)), _full((D, HEADS))],
        out_specs=(_rows((BE, DE)), _rows((BE, HEADS)),
                   pl.BlockSpec((1, 1), lambda i: (0, 0),
                                memory_space=pltpu.SMEM)),
        out_shape=(jax.ShapeDtypeStruct((E, DE), jnp.float32),
                   jax.ShapeDtypeStruct((E, HEADS), jnp.float32),
                   jax.ShapeDtypeStruct((1, 1), jnp.float32)),
        scratch_shapes=[pltpu.SMEM((1,), jnp.float32)],
    )(xsd, xsd, e, init_e, lw['we'][:D], lw['we'][D:2 * D], lw['we'][2 * D:],
      lw['be'].reshape(1, -1), lw['wq'], lw['wk'], lw['web'], sel)


def _edge_stage_c(xsd, new_e, logits, gmax, lw, rep4, p4to128):
    def body(xs_r, ne_r, lg_r, gm_r, wmx_r, wme_r, bm_r, rep_r, p128_r,
             em_o, ep_o):
        pre = (jnp.dot(xs_r[...], wmx_r[...], preferred_element_type=jnp.float32)
               + jnp.dot(ne_r[...], wme_r[...], preferred_element_type=jnp.float32)
               + bm_r[...])
        msg = _leaky(pre)
        ex = jnp.exp(lg_r[...] - gm_r[0, 0] + 40.0)
        exb = jnp.dot(ex, rep_r[...], preferred_element_type=jnp.float32)
        em_o[...] = exb * msg
        ep_o[...] = jnp.dot(ex, p128_r[...], preferred_element_type=jnp.float32)

    return pl.pallas_call(
        body,
        grid=(E // BE,),
        in_specs=[_rows((BE, D)), _rows((BE, DE)), _rows((BE, HEADS)),
                  pl.BlockSpec((1, 1), lambda i: (0, 0),
                               memory_space=pltpu.SMEM),
                  _full((D, D)), _full((DE, D)), _full((1, D)),
                  _full((HEADS, D)), _full((HEADS, D))],
        out_specs=(_rows((BE, D)), _rows((BE, D))),
        out_shape=(jax.ShapeDtypeStruct((E, D), jnp.float32),
                   jax.ShapeDtypeStruct((E, D), jnp.float32)),
    )(xsd, new_e, logits, gmax, lw['wm'][:D], lw['wm'][D:],
      lw['bm'].reshape(1, -1), rep4, p4to128)


def _node_stage(x, init_x, num0, num1, den0, den1, lw, rep128, dec=None):
    has_dec = dec is not None

    def body(*refs):
        if has_dec:
            (x_r, ix_r, n0_r, n1_r, d0_r, d1_r, wnx_r, wna_r, bn_r, rep_r,
             dw1_r, db1_r, dw2_r, db2_r, dw3_r, db3_r, o_r) = refs
        else:
            (x_r, ix_r, n0_r, n1_r, d0_r, d1_r, wnx_r, wna_r, bn_r, rep_r,
             o_r) = refs
        den = jnp.dot(d0_r[...] + d1_r[...], rep_r[...],
                      preferred_element_type=jnp.float32)
        num = n0_r[...] + n1_r[...]
        agg = jnp.where(den > 0, num / den, 0.0)
        pre = (jnp.dot(x_r[...], wnx_r[...], preferred_element_type=jnp.float32)
               + jnp.dot(agg, wna_r[...], preferred_element_type=jnp.float32)
               + bn_r[...])
        nx = _leaky(pre) + ix_r[...]
        if has_dec:
            h = _leaky(jnp.dot(nx, dw1_r[...], preferred_element_type=jnp.float32) + db1_r[...])
            h = _leaky(jnp.dot(h, dw2_r[...], preferred_element_type=jnp.float32) + db2_r[...])
            o_r[...] = jnp.dot(h, dw3_r[...], preferred_element_type=jnp.float32) + db3_r[...]
        else:
            o_r[...] = nx

    ins = [_rows((BN, D)), _rows((BN, D)), _rows((BN, D)), _rows((BN, D)),
           _rows((BN, D)), _rows((BN, D)), _full((D, D)), _full((D, D)),
           _full((1, D)), _full((D, D))]
    args = [x, init_x, num0, num1, den0, den1, lw['wn'][:D], lw['wn'][D:],
            lw['bn'].reshape(1, -1), rep128]
    if has_dec:
        ins += [_full((D, D)), _full((1, D)), _full((D, D)), _full((1, D)),
                _full((D, 1)), _full((1, 1))]
        args += [dec['w1'], dec['b1'].reshape(1, -1), dec['w2'],
                 dec['b2'].reshape(1, -1), dec['w3'], dec['b3'].reshape(1, -1)]
        out_spec = _rows((BN, 1))
        out_shape = jax.ShapeDtypeStruct((N, 1), jnp.float32)
    else:
        out_spec = _rows((BN, D))
        out_shape = jax.ShapeDtypeStruct((N, D), jnp.float32)

    return pl.pallas_call(
        body,
        grid=(N // BN,),
        in_specs=ins,
        out_specs=out_spec,
        out_shape=out_shape,
    )(*args)


def kernel(x, edge_index, edge_attr, params):
    p = params
    src = edge_index[0].astype(jnp.int32)
    dst = edge_index[1].astype(jnp.int32)
    sd2d = jnp.concatenate([src.reshape(-1, CH), dst.reshape(-1, CH)], axis=0)

    hd_ids = jnp.arange(D, dtype=jnp.int32) // HD
    sel = (hd_ids[:, None] == jnp.arange(HEADS)[None, :]).astype(jnp.float32)
    rep4 = sel.T
    p4to128 = jnp.eye(HEADS, D, dtype=jnp.float32)
    rep128 = jnp.zeros((D, D), jnp.float32).at[:HEADS].set(rep4)
    zc = jnp.zeros((CH, D), jnp.float32)

    init_x = _node_encoder(x, p['ne_w1'], p['ne_b1'], p['ne_w2'], p['ne_b2'])
    init_e = _edge_encoder(edge_attr, p['ee_w1'], p['ee_b1'],
                           p['ee_w2'], p['ee_b2'])

    cx, ce = init_x, init_e
    dec = {'w1': p['dec_w1'], 'b1': p['dec_b1'], 'w2': p['dec_w2'],
           'b2': p['dec_b2'], 'w3': p['dec_w3'], 'b3': p['dec_b3']}

    for li, lw in enumerate(p['layers']):
        xsd = _sc_gather(cx, sd2d)
        new_e, logits, gmax = _edge_stage_b(xsd, ce, init_e, lw, sel)
        exmsg, expad = _edge_stage_c(xsd, new_e, logits, gmax, lw, rep4,
                                     p4to128)
        num = _sc_scatter(exmsg, sd2d, zc)
        den = _sc_scatter(expad, sd2d, zc)
        last = li == len(p['layers']) - 1
        out = _node_stage(cx, init_x, num[:N], num[NP:NP + N], den[:N],
                          den[NP:NP + N], lw, rep128,
                          dec=dec if last else None)
        if last:
            return out
        cx, ce = out, new_e
